# Initial kernel scaffold; baseline (speedup 1.0000x reference)
import functools

import jax
import jax.numpy as jnp
from jax import lax
from jax.experimental import pallas as pl
from jax.experimental.pallas import tpu as pltpu
from jax.experimental.pallas import tpu_sc as plsc

N = 10000
D = 128
E = 320000
NC = 2
NS = 16
NW = NC * NS
EPW = E // NW
G = 80
NCH = EPW // G
RPT = N // NS
ZR = 25


def _sc_aggregate(y, src_r, dst_r):
    mesh = plsc.VectorSubcoreMesh(core_axis_name="c", subcore_axis_name="s")

    @functools.partial(
        pl.kernel,
        mesh=mesh,
        out_type=jax.ShapeDtypeStruct((NC, N, D), jnp.float32),
        scratch_types=[
            pltpu.VMEM((NCH, G), jnp.int32),
            pltpu.VMEM((NCH, G), jnp.int32),
            pltpu.VMEM((G, D), jnp.float32),
            pltpu.VMEM((ZR, D), jnp.float32),
            pltpu.VMEM_SHARED((N, D), jnp.float32),
            pltpu.SemaphoreType.DMA,
        ],
    )
    def k(y_hbm, src_hbm, dst_hbm, out_hbm, src_v, dst_v, gbuf, zbuf, acc, sem):
        cid = lax.axis_index("c")
        sid = lax.axis_index("s")
        wid = cid * NS + sid

        def zrow(r, carry):
            for kk in range(D // 16):
                zbuf[r, pl.ds(kk * 16, 16)] = jnp.zeros((16,), jnp.float32)
            return carry

        lax.fori_loop(0, ZR, zrow, 0)
        for t in range(RPT // ZR):
            pltpu.sync_copy(zbuf, acc.at[pl.ds(sid * RPT + t * ZR, ZR)])

        pltpu.sync_copy(src_hbm.at[wid], src_v)
        pltpu.sync_copy(dst_hbm.at[wid], dst_v)
        plsc.subcore_barrier()

        def body(j, carry):
            pltpu.async_copy(y_hbm.at[src_v.at[j]], gbuf, sem).wait()
            pltpu.sync_copy(gbuf, acc.at[dst_v.at[j]], add=True)
            return carry

        lax.fori_loop(0, NCH, body, 0)
        plsc.subcore_barrier()
        pltpu.sync_copy(acc.at[pl.ds(sid * RPT, RPT)],
                        out_hbm.at[cid, pl.ds(sid * RPT, RPT)])

    return k(y, src_r, dst_r)


BR = 2000


def _tc_first(x, wr, wc, br):
    def body(x_ref, wr_ref, wc_ref, br_ref, y_ref, z_ref):
        h = x_ref[...]
        y_ref[...] = jnp.dot(h, wr_ref[...], preferred_element_type=jnp.float32)
        z_ref[...] = (jnp.dot(h, wc_ref[...], preferred_element_type=jnp.float32)
                      + br_ref[...])

    return pl.pallas_call(
        body,
        grid=(N // BR,),
        in_specs=[
            pl.BlockSpec((BR, D), lambda i: (i, 0)),
            pl.BlockSpec((D, D), lambda i: (0, 0)),
            pl.BlockSpec((D, D), lambda i: (0, 0)),
            pl.BlockSpec((1, D), lambda i: (0, 0)),
        ],
        out_specs=[
            pl.BlockSpec((BR, D), lambda i: (i, 0)),
            pl.BlockSpec((BR, D), lambda i: (i, 0)),
        ],
        out_shape=[
            jax.ShapeDtypeStruct((N, D), jnp.float32),
            jax.ShapeDtypeStruct((N, D), jnp.float32),
        ],
    )(x, wr, wc, br)


def _tc_mid(a0, a1, z, wr, wc, br):
    def body(a0_ref, a1_ref, z_ref, wr_ref, wc_ref, br_ref, y_ref, z2_ref):
        h = jnp.maximum(a0_ref[...] + a1_ref[...] + z_ref[...], 0.0)
        y_ref[...] = jnp.dot(h, wr_ref[...], preferred_element_type=jnp.float32)
        z2_ref[...] = (jnp.dot(h, wc_ref[...], preferred_element_type=jnp.float32)
                       + br_ref[...])

    return pl.pallas_call(
        body,
        grid=(N // BR,),
        in_specs=[
            pl.BlockSpec((BR, D), lambda i: (i, 0)),
            pl.BlockSpec((BR, D), lambda i: (i, 0)),
            pl.BlockSpec((BR, D), lambda i: (i, 0)),
            pl.BlockSpec((D, D), lambda i: (0, 0)),
            pl.BlockSpec((D, D), lambda i: (0, 0)),
            pl.BlockSpec((1, D), lambda i: (0, 0)),
        ],
        out_specs=[
            pl.BlockSpec((BR, D), lambda i: (i, 0)),
            pl.BlockSpec((BR, D), lambda i: (i, 0)),
        ],
        out_shape=[
            jax.ShapeDtypeStruct((N, D), jnp.float32),
            jax.ShapeDtypeStruct((N, D), jnp.float32),
        ],
    )(a0, a1, z, wr, wc, br)


def _tc_final(a0, a1, z):
    def body(a0_ref, a1_ref, z_ref, o_ref):
        o_ref[...] = a0_ref[...] + a1_ref[...] + z_ref[...]

    return pl.pallas_call(
        body,
        grid=(N // BR,),
        in_specs=[
            pl.BlockSpec((BR, D), lambda i: (i, 0)),
            pl.BlockSpec((BR, D), lambda i: (i, 0)),
            pl.BlockSpec((BR, D), lambda i: (i, 0)),
        ],
        out_specs=pl.BlockSpec((BR, D), lambda i: (i, 0)),
        out_shape=jax.ShapeDtypeStruct((N, D), jnp.float32),
    )(a0, a1, z)


def kernel(x, edge_index, Wr1, br1, Wc1, Wr2, br2, Wc2, Wr3, br3, Wc3):
    src_r = edge_index[0].astype(jnp.int32).reshape(NW, NCH, G)
    dst_r = edge_index[1].astype(jnp.int32).reshape(NW, NCH, G)

    y, z = _tc_first(x, Wr1, Wc1, br1.reshape(1, D))
    a = _sc_aggregate(y, src_r, dst_r)
    y, z = _tc_mid(a[0], a[1], z, Wr2, Wc2, br2.reshape(1, D))
    a = _sc_aggregate(y, src_r, dst_r)
    y, z = _tc_mid(a[0], a[1], z, Wr3, Wc3, br3.reshape(1, D))
    a = _sc_aggregate(y, src_r, dst_r)
    return _tc_final(a[0], a[1], z)


# speedup vs baseline: 6.9700x; 6.9700x over previous
import functools

import jax
import jax.numpy as jnp
from jax import lax
from jax.experimental import pallas as pl
from jax.experimental.pallas import tpu as pltpu
from jax.experimental.pallas import tpu_sc as plsc

N = 10000
D = 128
E = 320000
NC = 2
NS = 16
NW = NC * NS
EPW = E // NW
G = 80
NCH = EPW // G
RPT = 632
N_PAD = RPT * NS
ZR = 8


def _sc_aggregate(y, src_r, dst_r):
    mesh = plsc.VectorSubcoreMesh(core_axis_name="c", subcore_axis_name="s")

    @functools.partial(
        pl.kernel,
        mesh=mesh,
        out_type=jax.ShapeDtypeStruct((NC, N_PAD, D), jnp.float32),
        scratch_types=[
            pltpu.VMEM((NCH, G), jnp.int32),
            pltpu.VMEM((NCH, G), jnp.int32),
            pltpu.VMEM((G, D), jnp.float32),
            pltpu.VMEM((ZR, D), jnp.float32),
            pltpu.VMEM_SHARED((N_PAD, D), jnp.float32),
            pltpu.SemaphoreType.DMA,
        ],
    )
    def k(y_hbm, src_hbm, dst_hbm, out_hbm, src_v, dst_v, gbuf, zbuf, acc, sem):
        cid = lax.axis_index("c")
        sid = lax.axis_index("s")
        wid = cid * NS + sid

        def zrow(r, carry):
            for kk in range(D // 16):
                zbuf[r, pl.ds(kk * 16, 16)] = jnp.zeros((16,), jnp.float32)
            return carry

        lax.fori_loop(0, ZR, zrow, 0)
        for t in range(RPT // ZR):
            pltpu.sync_copy(zbuf, acc.at[pl.ds(sid * RPT + t * ZR, ZR)])

        pltpu.sync_copy(src_hbm.at[wid], src_v)
        pltpu.sync_copy(dst_hbm.at[wid], dst_v)
        plsc.subcore_barrier()

        def body(j, carry):
            pltpu.async_copy(y_hbm.at[src_v.at[j]], gbuf, sem).wait()
            pltpu.sync_copy(gbuf, acc.at[dst_v.at[j]], add=True)
            return carry

        lax.fori_loop(0, NCH, body, 0)
        plsc.subcore_barrier()
        pltpu.sync_copy(acc.at[pl.ds(sid * RPT, RPT)],
                        out_hbm.at[cid, pl.ds(sid * RPT, RPT)])

    return k(y, src_r, dst_r)


BR = 2000


def _tc_first(x, wr, wc, br):
    def body(x_ref, wr_ref, wc_ref, br_ref, y_ref, z_ref):
        h = x_ref[...]
        y_ref[...] = jnp.dot(h, wr_ref[...], preferred_element_type=jnp.float32)
        z_ref[...] = (jnp.dot(h, wc_ref[...], preferred_element_type=jnp.float32)
                      + br_ref[...])

    return pl.pallas_call(
        body,
        grid=(N // BR,),
        in_specs=[
            pl.BlockSpec((BR, D), lambda i: (i, 0)),
            pl.BlockSpec((D, D), lambda i: (0, 0)),
            pl.BlockSpec((D, D), lambda i: (0, 0)),
            pl.BlockSpec((1, D), lambda i: (0, 0)),
        ],
        out_specs=[
            pl.BlockSpec((BR, D), lambda i: (i, 0)),
            pl.BlockSpec((BR, D), lambda i: (i, 0)),
        ],
        out_shape=[
            jax.ShapeDtypeStruct((N, D), jnp.float32),
            jax.ShapeDtypeStruct((N, D), jnp.float32),
        ],
    )(x, wr, wc, br)


def _tc_mid(a0, a1, z, wr, wc, br):
    def body(a0_ref, a1_ref, z_ref, wr_ref, wc_ref, br_ref, y_ref, z2_ref):
        h = jnp.maximum(a0_ref[...] + a1_ref[...] + z_ref[...], 0.0)
        y_ref[...] = jnp.dot(h, wr_ref[...], preferred_element_type=jnp.float32)
        z2_ref[...] = (jnp.dot(h, wc_ref[...], preferred_element_type=jnp.float32)
                       + br_ref[...])

    return pl.pallas_call(
        body,
        grid=(N // BR,),
        in_specs=[
            pl.BlockSpec((BR, D), lambda i: (i, 0)),
            pl.BlockSpec((BR, D), lambda i: (i, 0)),
            pl.BlockSpec((BR, D), lambda i: (i, 0)),
            pl.BlockSpec((D, D), lambda i: (0, 0)),
            pl.BlockSpec((D, D), lambda i: (0, 0)),
            pl.BlockSpec((1, D), lambda i: (0, 0)),
        ],
        out_specs=[
            pl.BlockSpec((BR, D), lambda i: (i, 0)),
            pl.BlockSpec((BR, D), lambda i: (i, 0)),
        ],
        out_shape=[
            jax.ShapeDtypeStruct((N, D), jnp.float32),
            jax.ShapeDtypeStruct((N, D), jnp.float32),
        ],
    )(a0, a1, z, wr, wc, br)


def _tc_final(a0, a1, z):
    def body(a0_ref, a1_ref, z_ref, o_ref):
        o_ref[...] = a0_ref[...] + a1_ref[...] + z_ref[...]

    return pl.pallas_call(
        body,
        grid=(N // BR,),
        in_specs=[
            pl.BlockSpec((BR, D), lambda i: (i, 0)),
            pl.BlockSpec((BR, D), lambda i: (i, 0)),
            pl.BlockSpec((BR, D), lambda i: (i, 0)),
        ],
        out_specs=pl.BlockSpec((BR, D), lambda i: (i, 0)),
        out_shape=jax.ShapeDtypeStruct((N, D), jnp.float32),
    )(a0, a1, z)


def kernel(x, edge_index, Wr1, br1, Wc1, Wr2, br2, Wc2, Wr3, br3, Wc3):
    src_r = edge_index[0].astype(jnp.int32).reshape(NW, NCH, G)
    dst_r = edge_index[1].astype(jnp.int32).reshape(NW, NCH, G)

    y, z = _tc_first(x, Wr1, Wc1, br1.reshape(1, D))
    a = _sc_aggregate(y, src_r, dst_r)
    y, z = _tc_mid(a[0], a[1], z, Wr2, Wc2, br2.reshape(1, D))
    a = _sc_aggregate(y, src_r, dst_r)
    y, z = _tc_mid(a[0], a[1], z, Wr3, Wc3, br3.reshape(1, D))
    a = _sc_aggregate(y, src_r, dst_r)
    return _tc_final(a[0], a[1], z)
